# baseline (speedup 1.0000x reference)
import functools

import jax
import jax.numpy as jnp
from jax import lax
from jax.experimental import pallas as pl
from jax.experimental.pallas import tpu as pltpu
from jax.experimental.pallas import tpu_sc as plsc

N = 10000
E = 320000
KEYSPACE = N * N
NC = 2
NS = 16
NW = NC * NS
L = 16
SPAN = KEYSPACE // NW
WORDS = ((SPAN + 31) // 32 + L - 1) // L * L
CHUNK = 3200
ROUNDS = E // CHUNK
IN_VECS = CHUNK // L
CAP = 12000

KB = 64000
KG = E // KB


def _keys_tc(edges1, edges2):
    def body(a_ref, b_ref, k1_ref, k2_ref):
        k1_ref[...] = a_ref[0:1, :] * N + a_ref[1:2, :]
        k2_ref[...] = b_ref[0:1, :] * N + b_ref[1:2, :]

    k1, k2 = pl.pallas_call(
        body,
        grid=(KG,),
        in_specs=[
            pl.BlockSpec((2, KB), lambda i: (0, i)),
            pl.BlockSpec((2, KB), lambda i: (0, i)),
        ],
        out_specs=[
            pl.BlockSpec((1, KB), lambda i: (0, i)),
            pl.BlockSpec((1, KB), lambda i: (0, i)),
        ],
        out_shape=[
            jax.ShapeDtypeStruct((1, E), jnp.int32),
            jax.ShapeDtypeStruct((1, E), jnp.int32),
        ],
    )(edges1, edges2)
    return k1.reshape(E), k2.reshape(E)


def _zero_bitmap(bm):
    zeros = jnp.zeros((L,), jnp.int32)

    def body(i, carry):
        bm[pl.ds(i * L, L)] = zeros
        return carry

    lax.fori_loop(0, WORDS // L, body, 0, unroll=8)


def _drain(cbuf, bm, cnt_ref, fill):
    lanes = lax.broadcasted_iota(jnp.int32, (L,), 0)

    def body(j, carry):
        valid = lanes < (fill - j * L)
        rs = cbuf[pl.ds(j * L, L)]
        rs = jnp.where(valid, rs, 0)
        w = lax.shift_right_logical(rs, 5)
        bit = lax.shift_left(jnp.int32(1), rs & 31)
        _, uniq = plsc.scan_count(rs, mask=valid)
        uniq = uniq & valid
        old = plsc.load_gather(bm, [w], mask=uniq)
        elig = uniq & ((old & bit) == 0)
        _, sel = plsc.scan_count(w, mask=elig)
        sel = sel & elig
        plsc.addupdate_scatter(bm, [w], bit, mask=sel)
        left = elig & jnp.logical_not(sel)

        @pl.when(jnp.any(left))
        def _():
            def cond(rem):
                return jnp.any(rem)

            def rmw(rem):
                _, s = plsc.scan_count(w, mask=rem)
                s = s & rem
                plsc.addupdate_scatter(bm, [w], bit, mask=s)
                return rem & jnp.logical_not(s)

            lax.while_loop(cond, rmw, left)

        cnt_ref[...] = cnt_ref[...] + jnp.where(elig, 1, 0)
        return carry

    nit = lax.div(fill + (L - 1), L)
    lax.fori_loop(0, nit, body, 0, unroll=False)


def _scan_stream(keys_hbm, bm, cnt_ref, bufs, sems, cbuf, lo):
    def start(g, buf, sem):
        pltpu.async_copy(keys_hbm.at[pl.ds(g * CHUNK, CHUNK)], buf, sem)

    def wait(buf, sem):
        src = keys_hbm.at[pl.ds(0, CHUNK)]
        pltpu.make_async_copy(src, buf, sem).wait()

    start(0, bufs[0], sems[0])
    ones = jnp.ones((L,), jnp.int32)
    zeros = jnp.zeros((L,), jnp.int32)

    def compact_chunk(buf, fill):
        def vec_body(i, fill):
            k = buf[pl.ds(i * L, L)]
            r = k - lo
            inm = plsc.bitcast(r, jnp.uint32) < jnp.uint32(SPAN)
            rank = plsc.cumsum(jnp.where(inm, ones, zeros))
            idx = fill + rank - 1
            plsc.store_scatter(cbuf, [idx], r, mask=inm)
            pc = plsc.all_reduce_population_count(inm)
            return fill + pc

        return lax.fori_loop(0, IN_VECS, vec_body, fill, unroll=4)

    def chunk_pair(gg, fill):
        for b in (0, 1):
            g = 2 * gg + b
            wait(bufs[b], sems[b])

            @pl.when(g + 1 < ROUNDS)
            def _():
                start(g + 1, bufs[1 - b], sems[1 - b])

            def no_drain(fill):
                return fill

            def do_drain(fill):
                _drain(cbuf, bm, cnt_ref, fill[0])
                return jnp.zeros((L,), jnp.int32)

            fill = lax.cond(fill[0] > CAP - CHUNK, do_drain, no_drain, fill)
            fill = compact_chunk(bufs[b], fill)
        return fill

    fill = lax.fori_loop(0, ROUNDS // 2, chunk_pair,
                         jnp.zeros((L,), jnp.int32), unroll=False)
    _drain(cbuf, bm, cnt_ref, fill[0])


def _sc_counts(keys1, keys2):
    mesh = plsc.VectorSubcoreMesh(
        core_axis_name="c", subcore_axis_name="s", num_cores=NC,
        num_subcores=NS)

    @functools.partial(
        pl.kernel,
        out_type=jax.ShapeDtypeStruct((3, NW, L), jnp.int32),
        mesh=mesh,
        scratch_types=[
            pltpu.VMEM((WORDS,), jnp.int32),
            pltpu.VMEM((CAP + CHUNK + L,), jnp.int32),
            pltpu.VMEM((CHUNK,), jnp.int32),
            pltpu.VMEM((CHUNK,), jnp.int32),
            pltpu.VMEM((L,), jnp.int32),
            pltpu.VMEM((L,), jnp.int32),
            pltpu.VMEM((L,), jnp.int32),
            pltpu.SemaphoreType.DMA,
            pltpu.SemaphoreType.DMA,
        ],
        compiler_params=pltpu.CompilerParams(needs_layout_passes=False),
    )
    def k(k1_hbm, k2_hbm, out_hbm, bm, cbuf, b0, b1,
          c1_ref, cu_ref, c2_ref, sem0, sem1):
        wid = lax.axis_index("s") * NC + lax.axis_index("c")
        lo = wid * SPAN
        bufs = (b0, b1)
        sems = (sem0, sem1)
        zero = jnp.zeros((L,), jnp.int32)
        c1_ref[...] = zero
        cu_ref[...] = zero
        c2_ref[...] = zero
        _zero_bitmap(bm)
        _scan_stream(k1_hbm, bm, c1_ref, bufs, sems, cbuf, lo)
        _scan_stream(k2_hbm, bm, cu_ref, bufs, sems, cbuf, lo)
        _zero_bitmap(bm)
        _scan_stream(k2_hbm, bm, c2_ref, bufs, sems, cbuf, lo)
        for idx, ref in ((0, c1_ref), (1, cu_ref), (2, c2_ref)):
            pltpu.sync_copy(ref, out_hbm.at[idx, wid])

    return k(keys1, keys2)


def _tc_finish(counts):
    def body(c_ref, o_ref):
        c = c_ref[...].astype(jnp.float32)
        s1 = jnp.sum(c[0])
        su = jnp.sum(c[1])
        s2 = jnp.sum(c[2])
        o_ref[...] = jnp.reshape((s2 - su) / (s1 + su), (1, 1))

    out = pl.pallas_call(
        body,
        out_shape=jax.ShapeDtypeStruct((1, 1), jnp.float32),
    )(counts)
    return out.reshape(())


def kernel(edges1, edges2, num_nodes):
    del num_nodes
    keys1, keys2 = _keys_tc(edges1, edges2)
    counts = _sc_counts(keys1, keys2)
    return _tc_finish(counts)


# speedup vs baseline: 1.4536x; 1.4536x over previous
import functools

import jax
import jax.numpy as jnp
from jax import lax
from jax.experimental import pallas as pl
from jax.experimental.pallas import tpu as pltpu
from jax.experimental.pallas import tpu_sc as plsc

N = 10000
E = 320000
KEYSPACE = N * N
NC = 2
NS = 16
NW = NC * NS
L = 16
SPAN = KEYSPACE // NW
WORDS = ((SPAN + 31) // 32 + L - 1) // L * L
CHUNK = 3200
ROUNDS = E // CHUNK
NB = 8
IN_GRPS = CHUNK // (NB * L)
CAPB = 1900
CBUF = CAPB + CHUNK // NB + L

KB = 64000
KG = E // KB


def _keys_tc(edges1, edges2):
    def body(a_ref, b_ref, k1_ref, k2_ref):
        k1_ref[...] = a_ref[0:1, :] * N + a_ref[1:2, :]
        k2_ref[...] = b_ref[0:1, :] * N + b_ref[1:2, :]

    k1, k2 = pl.pallas_call(
        body,
        grid=(KG,),
        in_specs=[
            pl.BlockSpec((2, KB), lambda i: (0, i)),
            pl.BlockSpec((2, KB), lambda i: (0, i)),
        ],
        out_specs=[
            pl.BlockSpec((1, KB), lambda i: (0, i)),
            pl.BlockSpec((1, KB), lambda i: (0, i)),
        ],
        out_shape=[
            jax.ShapeDtypeStruct((1, E), jnp.int32),
            jax.ShapeDtypeStruct((1, E), jnp.int32),
        ],
    )(edges1, edges2)
    return k1.reshape(E), k2.reshape(E)


def _zero_bitmap(bm):
    zeros = jnp.zeros((L,), jnp.int32)

    def body(i, carry):
        bm[pl.ds(i * L, L)] = zeros
        return carry

    lax.fori_loop(0, WORDS // L, body, 0, unroll=8)


def _drain(cbuf, bm, cnt_ref, fill):
    lanes = lax.broadcasted_iota(jnp.int32, (L,), 0)

    def body(j, carry):
        valid = lanes < (fill - j * L)
        rs = cbuf[pl.ds(j * L, L)]
        rs = jnp.where(valid, rs, 0)
        w = lax.shift_right_logical(rs, 5)
        bit = lax.shift_left(jnp.int32(1), rs & 31)
        _, uniq = plsc.scan_count(rs, mask=valid)
        uniq = uniq & valid
        old = plsc.load_gather(bm, [w], mask=uniq)
        elig = uniq & ((old & bit) == 0)
        _, sel = plsc.scan_count(w, mask=elig)
        sel = sel & elig
        plsc.addupdate_scatter(bm, [w], bit, mask=sel)
        left = elig & jnp.logical_not(sel)

        @pl.when(jnp.any(left))
        def _():
            def cond(rem):
                return jnp.any(rem)

            def rmw(rem):
                _, s = plsc.scan_count(w, mask=rem)
                s = s & rem
                plsc.addupdate_scatter(bm, [w], bit, mask=s)
                return rem & jnp.logical_not(s)

            lax.while_loop(cond, rmw, left)

        cnt_ref[...] = cnt_ref[...] + jnp.where(elig, 1, 0)
        return carry

    nit = lax.div(fill + (L - 1), L)
    lax.fori_loop(0, nit, body, 0, unroll=False)


def _scan_stream(keys_hbm, bm, cnt_ref, bufs, sems, cbufs, lo):
    def start(g, buf, sem):
        pltpu.async_copy(keys_hbm.at[pl.ds(g * CHUNK, CHUNK)], buf, sem)

    def wait(buf, sem):
        src = keys_hbm.at[pl.ds(0, CHUNK)]
        pltpu.make_async_copy(src, buf, sem).wait()

    start(0, bufs[0], sems[0])

    def compact_chunk(buf, fills):
        def grp_body(i, fills):
            new = []
            for b in range(NB):
                f = fills[b]
                k = buf[pl.ds((i * NB + b) * L, L)]
                r = k - lo
                inm = plsc.bitcast(r, jnp.uint32) < jnp.uint32(SPAN)
                plsc.store_compressed(cbufs[b].at[pl.ds(f, L)], r,
                                      mask=inm)
                pc = plsc.all_reduce_population_count(inm)
                new.append(f + pc[0])
            return tuple(new)

        return lax.fori_loop(0, IN_GRPS, grp_body, fills, unroll=2)

    def chunk_pair(gg, fills):
        for b in (0, 1):
            g = 2 * gg + b
            wait(bufs[b], sems[b])

            @pl.when(g + 1 < ROUNDS)
            def _():
                start(g + 1, bufs[1 - b], sems[1 - b])

            def no_drain(fills):
                return fills

            def do_drain(fills):
                for j in range(NB):
                    _drain(cbufs[j], bm, cnt_ref, fills[j])
                return (jnp.int32(0),) * NB

            mx = fills[0]
            for j in range(1, NB):
                mx = jnp.maximum(mx, fills[j])
            fills = lax.cond(mx > CAPB - CHUNK // NB, do_drain, no_drain,
                             fills)
            fills = compact_chunk(bufs[b], fills)
        return fills

    fills = lax.fori_loop(0, ROUNDS // 2, chunk_pair,
                          (jnp.int32(0),) * NB, unroll=False)
    for j in range(NB):
        _drain(cbufs[j], bm, cnt_ref, fills[j])


def _sc_counts(keys1, keys2):
    mesh = plsc.VectorSubcoreMesh(
        core_axis_name="c", subcore_axis_name="s", num_cores=NC,
        num_subcores=NS)

    @functools.partial(
        pl.kernel,
        out_type=jax.ShapeDtypeStruct((3, NW, L), jnp.int32),
        mesh=mesh,
        scratch_types=[
            pltpu.VMEM((WORDS,), jnp.int32),
        ] + [pltpu.VMEM((CBUF,), jnp.int32) for _ in range(NB)] + [
            pltpu.VMEM((CHUNK,), jnp.int32),
            pltpu.VMEM((CHUNK,), jnp.int32),
            pltpu.VMEM((L,), jnp.int32),
            pltpu.VMEM((L,), jnp.int32),
            pltpu.VMEM((L,), jnp.int32),
            pltpu.SemaphoreType.DMA,
            pltpu.SemaphoreType.DMA,
        ],
        compiler_params=pltpu.CompilerParams(needs_layout_passes=False),
    )
    def k(k1_hbm, k2_hbm, out_hbm, bm, cb0, cb1, cb2, cb3, cb4, cb5, cb6,
          cb7, b0, b1, c1_ref, cu_ref, c2_ref, sem0, sem1):
        wid = lax.axis_index("s") * NC + lax.axis_index("c")
        lo = wid * SPAN
        cbufs = (cb0, cb1, cb2, cb3, cb4, cb5, cb6, cb7)
        bufs = (b0, b1)
        sems = (sem0, sem1)
        zero = jnp.zeros((L,), jnp.int32)
        c1_ref[...] = zero
        cu_ref[...] = zero
        c2_ref[...] = zero
        _zero_bitmap(bm)
        _scan_stream(k1_hbm, bm, c1_ref, bufs, sems, cbufs, lo)
        _scan_stream(k2_hbm, bm, cu_ref, bufs, sems, cbufs, lo)
        _zero_bitmap(bm)
        _scan_stream(k2_hbm, bm, c2_ref, bufs, sems, cbufs, lo)
        for idx, ref in ((0, c1_ref), (1, cu_ref), (2, c2_ref)):
            pltpu.sync_copy(ref, out_hbm.at[idx, wid])

    return k(keys1, keys2)


def _tc_finish(counts):
    def body(c_ref, o_ref):
        c = c_ref[...].astype(jnp.float32)
        s1 = jnp.sum(c[0])
        su = jnp.sum(c[1])
        s2 = jnp.sum(c[2])
        o_ref[...] = jnp.reshape((s2 - su) / (s1 + su), (1, 1))

    out = pl.pallas_call(
        body,
        out_shape=jax.ShapeDtypeStruct((1, 1), jnp.float32),
    )(counts)
    return out.reshape(())


def kernel(edges1, edges2, num_nodes):
    del num_nodes
    keys1, keys2 = _keys_tc(edges1, edges2)
    counts = _sc_counts(keys1, keys2)
    return _tc_finish(counts)


# speedup vs baseline: 2.2603x; 1.5550x over previous
import functools

import jax
import jax.numpy as jnp
from jax import lax
from jax.experimental import pallas as pl
from jax.experimental.pallas import tpu as pltpu
from jax.experimental.pallas import tpu_sc as plsc

N = 10000
E = 320000
KEYSPACE = N * N
NC = 2
NS = 16
NW = NC * NS
L = 16
SPAN = KEYSPACE // NW
WORDS = ((SPAN + 31) // 32 + L - 1) // L * L
CHUNK = 3200
ROUNDS = E // CHUNK
IN_PAIRS = CHUNK // (2 * L)
CAPH = 6000
CBUF = 7808
SPILL = E + 4 * CBUF
SENT = -1

KB = 64000
KG = E // KB


def _keys_tc(edges1, edges2):
    def body(a_ref, b_ref, k1_ref, k2_ref):
        k1_ref[...] = a_ref[0:1, :] * N + a_ref[1:2, :]
        k2_ref[...] = b_ref[0:1, :] * N + b_ref[1:2, :]

    k1, k2 = pl.pallas_call(
        body,
        grid=(KG,),
        in_specs=[
            pl.BlockSpec((2, KB), lambda i: (0, i)),
            pl.BlockSpec((2, KB), lambda i: (0, i)),
        ],
        out_specs=[
            pl.BlockSpec((1, KB), lambda i: (0, i)),
            pl.BlockSpec((1, KB), lambda i: (0, i)),
        ],
        out_shape=[
            jax.ShapeDtypeStruct((1, E), jnp.int32),
            jax.ShapeDtypeStruct((1, E), jnp.int32),
        ],
    )(edges1, edges2)
    return k1.reshape(E), k2.reshape(E)


def _zero_bitmap(bm):
    zeros = jnp.zeros((L,), jnp.int32)

    def body(i, carry):
        bm[pl.ds(i * L, L)] = zeros
        return carry

    lax.fori_loop(0, WORDS // L, body, 0, unroll=8)


def _drain(cbuf, bm, cnt_ref, fill):
    lanes = lax.broadcasted_iota(jnp.int32, (L,), 0)

    def body(j, carry):
        rs = cbuf[pl.ds(j * L, L)]
        valid = (lanes < (fill - j * L)) & (rs != SENT)
        rs = jnp.where(valid, rs, 0)
        w = lax.shift_right_logical(rs, 5)
        bit = lax.shift_left(jnp.int32(1), rs & 31)
        _, uniq = plsc.scan_count(rs, mask=valid)
        uniq = uniq & valid
        old = plsc.load_gather(bm, [w], mask=uniq)
        elig = uniq & ((old & bit) == 0)
        _, sel = plsc.scan_count(w, mask=elig)
        sel = sel & elig
        plsc.addupdate_scatter(bm, [w], bit, mask=sel)
        left = elig & jnp.logical_not(sel)

        @pl.when(jnp.any(left))
        def _():
            def cond(rem):
                return jnp.any(rem)

            def rmw(rem):
                _, s = plsc.scan_count(w, mask=rem)
                s = s & rem
                plsc.addupdate_scatter(bm, [w], bit, mask=s)
                return rem & jnp.logical_not(s)

            lax.while_loop(cond, rmw, left)

        cnt_ref[...] = cnt_ref[...] + jnp.where(elig, 1, 0)
        return carry

    nit = lax.div(fill + (L - 1), L)
    lax.fori_loop(0, nit, body, 0, unroll=False)


def _scan_stream(keys_hbm, bm, cnt_ref, bufs, sems, cbufa, cbufb, lo,
                 spill_row, sentv):
    def start(g, buf, sem):
        pltpu.async_copy(keys_hbm.at[pl.ds(g * CHUNK, CHUNK)], buf, sem)

    def wait(buf, sem):
        src = keys_hbm.at[pl.ds(0, CHUNK)]
        pltpu.make_async_copy(src, buf, sem).wait()

    start(0, bufs[0], sems[0])

    def spill_and_drain(cbuf, fill, cur):
        if spill_row is None:
            _drain(cbuf, bm, cnt_ref, fill)
            return cur
        true16 = jnp.full((L,), True)
        for t in range(8):
            plsc.store_compressed(cbuf.at[pl.ds(fill + t * L, L)], sentv,
                                  mask=true16)
        dst = pl.multiple_of(spill_row[1] + cur, 128)
        pltpu.sync_copy(cbuf, spill_row[0].at[pl.ds(dst, CBUF)])
        _drain(cbuf, bm, cnt_ref, fill)
        return (cur + fill + 127) & ~127

    def compact_chunk(buf, fills):
        def pair_body(i, fills):
            fa, fb = fills
            ka = buf[pl.ds((2 * i) * L, L)]
            kb = buf[pl.ds((2 * i + 1) * L, L)]
            ra = ka - lo
            rb = kb - lo
            inma = plsc.bitcast(ra, jnp.uint32) < jnp.uint32(SPAN)
            inmb = plsc.bitcast(rb, jnp.uint32) < jnp.uint32(SPAN)
            plsc.store_compressed(cbufa.at[pl.ds(fa, L)], ra, mask=inma)
            plsc.store_compressed(cbufb.at[pl.ds(fb, L)], rb, mask=inmb)
            pca = plsc.all_reduce_population_count(inma)
            pcb = plsc.all_reduce_population_count(inmb)
            return fa + pca[0], fb + pcb[0]

        return lax.fori_loop(0, IN_PAIRS, pair_body, fills, unroll=4)

    def chunk_pair(gg, carry):
        for b in (0, 1):
            g = 2 * gg + b
            wait(bufs[b], sems[b])

            @pl.when(g + 1 < ROUNDS)
            def _():
                start(g + 1, bufs[1 - b], sems[1 - b])

            def no_drain(carry):
                return carry

            def do_drain(carry):
                fa, fb, cur = carry
                cur = spill_and_drain(cbufa, fa, cur)
                cur = spill_and_drain(cbufb, fb, cur)
                return jnp.int32(0), jnp.int32(0), cur

            fa, fb, cur = carry
            pred = jnp.maximum(fa, fb) > CAPH - CHUNK // 2
            carry = lax.cond(pred, do_drain, no_drain, (fa, fb, cur))
            fa, fb = compact_chunk(bufs[b], carry[:2])
            carry = (fa, fb, carry[2])
        return carry

    fa, fb, cur = lax.fori_loop(
        0, ROUNDS // 2, chunk_pair,
        (jnp.int32(0), jnp.int32(0), jnp.int32(0)), unroll=False)
    cur = spill_and_drain(cbufa, fa, cur)
    cur = spill_and_drain(cbufb, fb, cur)
    return cur


def _replay_spill(spill_row, total, bm, cnt_ref, bufs, sems):
    hbm, base = spill_row

    def start(j, buf, sem):
        off = pl.multiple_of(base + j * CHUNK, 128)
        pltpu.async_copy(hbm.at[pl.ds(off, CHUNK)], buf, sem)

    def wait(buf, sem):
        src = hbm.at[pl.ds(0, CHUNK)]
        pltpu.make_async_copy(src, buf, sem).wait()

    nchunks = lax.div(total + (CHUNK - 1), CHUNK)

    @pl.when(nchunks > 0)
    def _():
        start(0, bufs[0], sems[0])

        def chunk_pair(jj, carry):
            for b in (0, 1):
                j = 2 * jj + b

                @pl.when(j < nchunks)
                def _():
                    wait(bufs[b], sems[b])

                    @pl.when(j + 1 < nchunks)
                    def _():
                        start(j + 1, bufs[1 - b], sems[1 - b])

                    cnt = jnp.minimum(total - j * CHUNK, CHUNK)
                    _drain(bufs[b], bm, cnt_ref, cnt)

            return carry

        npairs = lax.div(nchunks + 1, 2)
        lax.fori_loop(0, npairs, chunk_pair, 0, unroll=False)


def _sc_counts(keys1, keys2):
    mesh = plsc.VectorSubcoreMesh(
        core_axis_name="c", subcore_axis_name="s", num_cores=NC,
        num_subcores=NS)

    @functools.partial(
        pl.kernel,
        out_type=[
            jax.ShapeDtypeStruct((3, NW, L), jnp.int32),
            jax.ShapeDtypeStruct((NW * SPILL,), jnp.int32),
        ],
        mesh=mesh,
        scratch_types=[
            pltpu.VMEM((WORDS,), jnp.int32),
            pltpu.VMEM((CBUF,), jnp.int32),
            pltpu.VMEM((CBUF,), jnp.int32),
            pltpu.VMEM((CHUNK,), jnp.int32),
            pltpu.VMEM((CHUNK,), jnp.int32),
            pltpu.VMEM((L,), jnp.int32),
            pltpu.VMEM((L,), jnp.int32),
            pltpu.VMEM((L,), jnp.int32),
            pltpu.SemaphoreType.DMA,
            pltpu.SemaphoreType.DMA,
        ],
        compiler_params=pltpu.CompilerParams(needs_layout_passes=False),
    )
    def k(k1_hbm, k2_hbm, out_hbm, spill_hbm, bm, cbufa, cbufb, b0, b1,
          c1_ref, cu_ref, c2_ref, sem0, sem1):
        wid = lax.axis_index("s") * NC + lax.axis_index("c")
        lo = wid * SPAN
        bufs = (b0, b1)
        sems = (sem0, sem1)
        zero = jnp.zeros((L,), jnp.int32)
        sentv = jnp.full((L,), SENT, jnp.int32)
        c1_ref[...] = zero
        cu_ref[...] = zero
        c2_ref[...] = zero
        spill_row = (spill_hbm, wid * SPILL)
        _zero_bitmap(bm)
        _scan_stream(k1_hbm, bm, c1_ref, bufs, sems, cbufa, cbufb, lo,
                     None, sentv)
        total = _scan_stream(k2_hbm, bm, cu_ref, bufs, sems, cbufa,
                             cbufb, lo, spill_row, sentv)
        _zero_bitmap(bm)
        _replay_spill(spill_row, total, bm, c2_ref, bufs, sems)
        for idx, ref in ((0, c1_ref), (1, cu_ref), (2, c2_ref)):
            pltpu.sync_copy(ref, out_hbm.at[idx, wid])

    return k(keys1, keys2)


def _tc_finish(counts):
    def body(c_ref, o_ref):
        c = c_ref[...].astype(jnp.float32)
        s1 = jnp.sum(c[0])
        su = jnp.sum(c[1])
        s2 = jnp.sum(c[2])
        o_ref[...] = jnp.reshape((s2 - su) / (s1 + su), (1, 1))

    out = pl.pallas_call(
        body,
        out_shape=jax.ShapeDtypeStruct((1, 1), jnp.float32),
    )(counts)
    return out.reshape(())


def kernel(edges1, edges2, num_nodes):
    del num_nodes
    keys1, keys2 = _keys_tc(edges1, edges2)
    counts, _ = _sc_counts(keys1, keys2)
    return _tc_finish(counts)


# speedup vs baseline: 2.2730x; 1.0056x over previous
import functools

import jax
import jax.numpy as jnp
from jax import lax
from jax.experimental import pallas as pl
from jax.experimental.pallas import tpu as pltpu
from jax.experimental.pallas import tpu_sc as plsc

N = 10000
E = 320000
KEYSPACE = N * N
NC = 2
NS = 16
NW = NC * NS
L = 16
SPAN = KEYSPACE // NW
WORDS = ((SPAN + 31) // 32 + L - 1) // L * L
CHUNK = 3200
ROUNDS = E // CHUNK
IN_PAIRS = CHUNK // (2 * L)
CAPH = 6000
CBUF = 7808
SPILL = E + 4 * CBUF
SENT = -1

KB = 64000
KG = E // KB


def _keys_tc(edges1, edges2):
    def body(a_ref, b_ref, k1_ref, k2_ref):
        k1_ref[...] = a_ref[0:1, :] * N + a_ref[1:2, :]
        k2_ref[...] = b_ref[0:1, :] * N + b_ref[1:2, :]

    k1, k2 = pl.pallas_call(
        body,
        grid=(KG,),
        in_specs=[
            pl.BlockSpec((2, KB), lambda i: (0, i)),
            pl.BlockSpec((2, KB), lambda i: (0, i)),
        ],
        out_specs=[
            pl.BlockSpec((1, KB), lambda i: (0, i)),
            pl.BlockSpec((1, KB), lambda i: (0, i)),
        ],
        out_shape=[
            jax.ShapeDtypeStruct((1, E), jnp.int32),
            jax.ShapeDtypeStruct((1, E), jnp.int32),
        ],
    )(edges1, edges2)
    return k1.reshape(E), k2.reshape(E)


def _zero_bitmap(bm):
    zeros = jnp.zeros((L,), jnp.int32)

    def body(i, carry):
        bm[pl.ds(i * L, L)] = zeros
        return carry

    lax.fori_loop(0, WORDS // L, body, 0, unroll=8)


def _drain(cbuf, bm, cnt_ref, fill):
    lanes = lax.broadcasted_iota(jnp.int32, (L,), 0)

    def body(j, carry):
        rs = cbuf[pl.ds(j * L, L)]
        valid = (lanes < (fill - j * L)) & (rs != SENT)
        rs = jnp.where(valid, rs, 0)
        w = lax.shift_right_logical(rs, 5)
        bit = lax.shift_left(jnp.int32(1), rs & 31)
        _, uniq = plsc.scan_count(rs, mask=valid)
        uniq = uniq & valid
        old = plsc.load_gather(bm, [w], mask=uniq)
        elig = uniq & ((old & bit) == 0)
        _, sel = plsc.scan_count(w, mask=elig)
        sel = sel & elig
        plsc.addupdate_scatter(bm, [w], bit, mask=sel)
        left = elig & jnp.logical_not(sel)

        @pl.when(jnp.any(left))
        def _():
            def cond(rem):
                return jnp.any(rem)

            def rmw(rem):
                _, s = plsc.scan_count(w, mask=rem)
                s = s & rem
                plsc.addupdate_scatter(bm, [w], bit, mask=s)
                return rem & jnp.logical_not(s)

            lax.while_loop(cond, rmw, left)

        cnt_ref[...] = cnt_ref[...] + jnp.where(elig, 1, 0)
        return carry

    nit = lax.div(fill + (L - 1), L)
    lax.fori_loop(0, nit, body, 0, unroll=False)


def _scan_stream(keys_hbm, bm, cnt_ref, bufs, sems, cbufa, cbufb, lo,
                 spill_row, sentv):
    def start(g, buf, sem):
        pltpu.async_copy(keys_hbm.at[pl.ds(g * CHUNK, CHUNK)], buf, sem)

    def wait(buf, sem):
        src = keys_hbm.at[pl.ds(0, CHUNK)]
        pltpu.make_async_copy(src, buf, sem).wait()

    start(0, bufs[0], sems[0])

    def spill_and_drain(cbuf, fill, cur):
        if spill_row is None:
            _drain(cbuf, bm, cnt_ref, fill)
            return cur
        true16 = jnp.full((L,), True)
        for t in range(8):
            plsc.store_compressed(cbuf.at[pl.ds(fill + t * L, L)], sentv,
                                  mask=true16)
        dst = pl.multiple_of(spill_row[1] + cur, 128)
        pltpu.sync_copy(cbuf, spill_row[0].at[pl.ds(dst, CBUF)])
        _drain(cbuf, bm, cnt_ref, fill)
        return (cur + fill + 127) & ~127

    def compact_chunk(buf, fills):
        def pair_body(i, fills):
            fa, fb = fills
            ka = buf[pl.ds((2 * i) * L, L)]
            kb = buf[pl.ds((2 * i + 1) * L, L)]
            ra = ka - lo
            rb = kb - lo
            inma = plsc.bitcast(ra, jnp.uint32) < jnp.uint32(SPAN)
            inmb = plsc.bitcast(rb, jnp.uint32) < jnp.uint32(SPAN)
            plsc.store_compressed(cbufa.at[pl.ds(fa, L)], ra, mask=inma)
            plsc.store_compressed(cbufb.at[pl.ds(fb, L)], rb, mask=inmb)
            pca = plsc.all_reduce_population_count(inma)
            pcb = plsc.all_reduce_population_count(inmb)
            return fa + pca[0], fb + pcb[0]

        return lax.fori_loop(0, IN_PAIRS, pair_body, fills, unroll=10)

    def chunk_pair(gg, carry):
        for b in (0, 1):
            g = 2 * gg + b
            wait(bufs[b], sems[b])

            @pl.when(g + 1 < ROUNDS)
            def _():
                start(g + 1, bufs[1 - b], sems[1 - b])

            def no_drain(carry):
                return carry

            def do_drain(carry):
                fa, fb, cur = carry
                cur = spill_and_drain(cbufa, fa, cur)
                cur = spill_and_drain(cbufb, fb, cur)
                return jnp.int32(0), jnp.int32(0), cur

            fa, fb, cur = carry
            pred = jnp.maximum(fa, fb) > CAPH - CHUNK // 2
            carry = lax.cond(pred, do_drain, no_drain, (fa, fb, cur))
            fa, fb = compact_chunk(bufs[b], carry[:2])
            carry = (fa, fb, carry[2])
        return carry

    fa, fb, cur = lax.fori_loop(
        0, ROUNDS // 2, chunk_pair,
        (jnp.int32(0), jnp.int32(0), jnp.int32(0)), unroll=False)
    cur = spill_and_drain(cbufa, fa, cur)
    cur = spill_and_drain(cbufb, fb, cur)
    return cur


def _replay_spill(spill_row, total, bm, cnt_ref, bufs, sems):
    hbm, base = spill_row

    def start(j, buf, sem):
        off = pl.multiple_of(base + j * CHUNK, 128)
        pltpu.async_copy(hbm.at[pl.ds(off, CHUNK)], buf, sem)

    def wait(buf, sem):
        src = hbm.at[pl.ds(0, CHUNK)]
        pltpu.make_async_copy(src, buf, sem).wait()

    nchunks = lax.div(total + (CHUNK - 1), CHUNK)

    @pl.when(nchunks > 0)
    def _():
        start(0, bufs[0], sems[0])

        def chunk_pair(jj, carry):
            for b in (0, 1):
                j = 2 * jj + b

                @pl.when(j < nchunks)
                def _():
                    wait(bufs[b], sems[b])

                    @pl.when(j + 1 < nchunks)
                    def _():
                        start(j + 1, bufs[1 - b], sems[1 - b])

                    cnt = jnp.minimum(total - j * CHUNK, CHUNK)
                    _drain(bufs[b], bm, cnt_ref, cnt)

            return carry

        npairs = lax.div(nchunks + 1, 2)
        lax.fori_loop(0, npairs, chunk_pair, 0, unroll=False)


def _sc_counts(keys1, keys2):
    mesh = plsc.VectorSubcoreMesh(
        core_axis_name="c", subcore_axis_name="s", num_cores=NC,
        num_subcores=NS)

    @functools.partial(
        pl.kernel,
        out_type=[
            jax.ShapeDtypeStruct((3, NW, L), jnp.int32),
            jax.ShapeDtypeStruct((NW * SPILL,), jnp.int32),
        ],
        mesh=mesh,
        scratch_types=[
            pltpu.VMEM((WORDS,), jnp.int32),
            pltpu.VMEM((CBUF,), jnp.int32),
            pltpu.VMEM((CBUF,), jnp.int32),
            pltpu.VMEM((CHUNK,), jnp.int32),
            pltpu.VMEM((CHUNK,), jnp.int32),
            pltpu.VMEM((L,), jnp.int32),
            pltpu.VMEM((L,), jnp.int32),
            pltpu.VMEM((L,), jnp.int32),
            pltpu.SemaphoreType.DMA,
            pltpu.SemaphoreType.DMA,
        ],
        compiler_params=pltpu.CompilerParams(needs_layout_passes=False),
    )
    def k(k1_hbm, k2_hbm, out_hbm, spill_hbm, bm, cbufa, cbufb, b0, b1,
          c1_ref, cu_ref, c2_ref, sem0, sem1):
        wid = lax.axis_index("s") * NC + lax.axis_index("c")
        lo = wid * SPAN
        bufs = (b0, b1)
        sems = (sem0, sem1)
        zero = jnp.zeros((L,), jnp.int32)
        sentv = jnp.full((L,), SENT, jnp.int32)
        c1_ref[...] = zero
        cu_ref[...] = zero
        c2_ref[...] = zero
        spill_row = (spill_hbm, wid * SPILL)
        _zero_bitmap(bm)
        _scan_stream(k1_hbm, bm, c1_ref, bufs, sems, cbufa, cbufb, lo,
                     None, sentv)
        total = _scan_stream(k2_hbm, bm, cu_ref, bufs, sems, cbufa,
                             cbufb, lo, spill_row, sentv)
        _zero_bitmap(bm)
        _replay_spill(spill_row, total, bm, c2_ref, bufs, sems)
        for idx, ref in ((0, c1_ref), (1, cu_ref), (2, c2_ref)):
            pltpu.sync_copy(ref, out_hbm.at[idx, wid])

    return k(keys1, keys2)


def _tc_finish(counts):
    def body(c_ref, o_ref):
        c = c_ref[...].astype(jnp.float32)
        s1 = jnp.sum(c[0])
        su = jnp.sum(c[1])
        s2 = jnp.sum(c[2])
        o_ref[...] = jnp.reshape((s2 - su) / (s1 + su), (1, 1))

    out = pl.pallas_call(
        body,
        out_shape=jax.ShapeDtypeStruct((1, 1), jnp.float32),
    )(counts)
    return out.reshape(())


def kernel(edges1, edges2, num_nodes):
    del num_nodes
    keys1, keys2 = _keys_tc(edges1, edges2)
    counts, _ = _sc_counts(keys1, keys2)
    return _tc_finish(counts)


# speedup vs baseline: 2.3871x; 1.0502x over previous
import functools

import jax
import jax.numpy as jnp
from jax import lax
from jax.experimental import pallas as pl
from jax.experimental.pallas import tpu as pltpu
from jax.experimental.pallas import tpu_sc as plsc

N = 10000
E = 320000
KEYSPACE = N * N
NC = 2
NS = 16
NW = NC * NS
L = 16
SPAN = KEYSPACE // NW
WORDS = ((SPAN + 31) // 32 + L - 1) // L * L
CHUNK = 3200
ROUNDS = E // CHUNK
IN_PAIRS = CHUNK // (2 * L)
CAPH = 6000
CBUF = 7808
SPILL = E + 4 * CBUF
SENT = -1

KB = 64000
KG = E // KB


def _keys_tc(edges1, edges2):
    def body(a_ref, b_ref, k1_ref, k2_ref):
        k1_ref[...] = a_ref[0:1, :] * N + a_ref[1:2, :]
        k2_ref[...] = b_ref[0:1, :] * N + b_ref[1:2, :]

    k1, k2 = pl.pallas_call(
        body,
        grid=(KG,),
        in_specs=[
            pl.BlockSpec((2, KB), lambda i: (0, i)),
            pl.BlockSpec((2, KB), lambda i: (0, i)),
        ],
        out_specs=[
            pl.BlockSpec((1, KB), lambda i: (0, i)),
            pl.BlockSpec((1, KB), lambda i: (0, i)),
        ],
        out_shape=[
            jax.ShapeDtypeStruct((1, E), jnp.int32),
            jax.ShapeDtypeStruct((1, E), jnp.int32),
        ],
    )(edges1, edges2)
    return k1.reshape(E), k2.reshape(E)


def _zero_bitmap(bm):
    zeros = jnp.zeros((L,), jnp.int32)

    def body(i, carry):
        bm[pl.ds(i * L, L)] = zeros
        return carry

    lax.fori_loop(0, WORDS // L, body, 0, unroll=8)


def _drain(cbuf, bm, cnt_ref, fill):
    lanes = lax.broadcasted_iota(jnp.int32, (L,), 0)

    def body(j, carry):
        rs = cbuf[pl.ds(j * L, L)]
        valid = (lanes < (fill - j * L)) & (rs != SENT)
        rs = jnp.where(valid, rs, 0)
        w = lax.shift_right_logical(rs, 5)
        bit = lax.shift_left(jnp.int32(1), rs & 31)
        old = plsc.load_gather(bm, [w], mask=valid)
        elig = valid & ((old & bit) == 0)
        _, sel = plsc.scan_count(w, mask=elig)
        sel = sel & elig
        plsc.addupdate_scatter(bm, [w], bit, mask=sel)
        cnt_ref[...] = cnt_ref[...] + jnp.where(sel, 1, 0)
        left = elig & jnp.logical_not(sel)

        @pl.when(jnp.any(left))
        def _():
            def cond(rem):
                return jnp.any(rem)

            def rmw(rem):
                o = plsc.load_gather(bm, [w], mask=rem)
                er = rem & ((o & bit) == 0)
                _, s = plsc.scan_count(w, mask=er)
                s = s & er
                plsc.addupdate_scatter(bm, [w], bit, mask=s)
                cnt_ref[...] = cnt_ref[...] + jnp.where(s, 1, 0)
                return er & jnp.logical_not(s)

            lax.while_loop(cond, rmw, left)

        return carry

    nit = lax.div(fill + (L - 1), L)
    lax.fori_loop(0, nit, body, 0, unroll=False)


def _scan_stream(keys_hbm, bm, cnt_ref, bufs, sems, cbufa, cbufb, lo,
                 spill_row, sentv):
    def start(g, buf, sem):
        pltpu.async_copy(keys_hbm.at[pl.ds(g * CHUNK, CHUNK)], buf, sem)

    def wait(buf, sem):
        src = keys_hbm.at[pl.ds(0, CHUNK)]
        pltpu.make_async_copy(src, buf, sem).wait()

    start(0, bufs[0], sems[0])

    def spill_and_drain(cbuf, fill, cur):
        if spill_row is None:
            _drain(cbuf, bm, cnt_ref, fill)
            return cur
        true16 = jnp.full((L,), True)
        for t in range(8):
            plsc.store_compressed(cbuf.at[pl.ds(fill + t * L, L)], sentv,
                                  mask=true16)
        dst = pl.multiple_of(spill_row[1] + cur, 128)
        pltpu.sync_copy(cbuf, spill_row[0].at[pl.ds(dst, CBUF)])
        _drain(cbuf, bm, cnt_ref, fill)
        return (cur + fill + 127) & ~127

    def compact_chunk(buf, fills):
        def pair_body(i, fills):
            fa, fb = fills
            ka = buf[pl.ds((2 * i) * L, L)]
            kb = buf[pl.ds((2 * i + 1) * L, L)]
            ra = ka - lo
            rb = kb - lo
            inma = plsc.bitcast(ra, jnp.uint32) < jnp.uint32(SPAN)
            inmb = plsc.bitcast(rb, jnp.uint32) < jnp.uint32(SPAN)
            plsc.store_compressed(cbufa.at[pl.ds(fa, L)], ra, mask=inma)
            plsc.store_compressed(cbufb.at[pl.ds(fb, L)], rb, mask=inmb)
            pca = plsc.all_reduce_population_count(inma)
            pcb = plsc.all_reduce_population_count(inmb)
            return fa + pca[0], fb + pcb[0]

        return lax.fori_loop(0, IN_PAIRS, pair_body, fills, unroll=10)

    def chunk_pair(gg, carry):
        for b in (0, 1):
            g = 2 * gg + b
            wait(bufs[b], sems[b])

            @pl.when(g + 1 < ROUNDS)
            def _():
                start(g + 1, bufs[1 - b], sems[1 - b])

            def no_drain(carry):
                return carry

            def do_drain(carry):
                fa, fb, cur = carry
                cur = spill_and_drain(cbufa, fa, cur)
                cur = spill_and_drain(cbufb, fb, cur)
                return jnp.int32(0), jnp.int32(0), cur

            fa, fb, cur = carry
            pred = jnp.maximum(fa, fb) > CAPH - CHUNK // 2
            carry = lax.cond(pred, do_drain, no_drain, (fa, fb, cur))
            fa, fb = compact_chunk(bufs[b], carry[:2])
            carry = (fa, fb, carry[2])
        return carry

    fa, fb, cur = lax.fori_loop(
        0, ROUNDS // 2, chunk_pair,
        (jnp.int32(0), jnp.int32(0), jnp.int32(0)), unroll=False)
    cur = spill_and_drain(cbufa, fa, cur)
    cur = spill_and_drain(cbufb, fb, cur)
    return cur


def _replay_spill(spill_row, total, bm, cnt_ref, bufs, sems):
    hbm, base = spill_row

    def start(j, buf, sem):
        off = pl.multiple_of(base + j * CHUNK, 128)
        pltpu.async_copy(hbm.at[pl.ds(off, CHUNK)], buf, sem)

    def wait(buf, sem):
        src = hbm.at[pl.ds(0, CHUNK)]
        pltpu.make_async_copy(src, buf, sem).wait()

    nchunks = lax.div(total + (CHUNK - 1), CHUNK)

    @pl.when(nchunks > 0)
    def _():
        start(0, bufs[0], sems[0])

        def chunk_pair(jj, carry):
            for b in (0, 1):
                j = 2 * jj + b

                @pl.when(j < nchunks)
                def _():
                    wait(bufs[b], sems[b])

                    @pl.when(j + 1 < nchunks)
                    def _():
                        start(j + 1, bufs[1 - b], sems[1 - b])

                    cnt = jnp.minimum(total - j * CHUNK, CHUNK)
                    _drain(bufs[b], bm, cnt_ref, cnt)

            return carry

        npairs = lax.div(nchunks + 1, 2)
        lax.fori_loop(0, npairs, chunk_pair, 0, unroll=False)


def _sc_counts(keys1, keys2):
    mesh = plsc.VectorSubcoreMesh(
        core_axis_name="c", subcore_axis_name="s", num_cores=NC,
        num_subcores=NS)

    @functools.partial(
        pl.kernel,
        out_type=[
            jax.ShapeDtypeStruct((3, NW, L), jnp.int32),
            jax.ShapeDtypeStruct((NW * SPILL,), jnp.int32),
        ],
        mesh=mesh,
        scratch_types=[
            pltpu.VMEM((WORDS,), jnp.int32),
            pltpu.VMEM((CBUF,), jnp.int32),
            pltpu.VMEM((CBUF,), jnp.int32),
            pltpu.VMEM((CHUNK,), jnp.int32),
            pltpu.VMEM((CHUNK,), jnp.int32),
            pltpu.VMEM((L,), jnp.int32),
            pltpu.VMEM((L,), jnp.int32),
            pltpu.VMEM((L,), jnp.int32),
            pltpu.SemaphoreType.DMA,
            pltpu.SemaphoreType.DMA,
        ],
        compiler_params=pltpu.CompilerParams(needs_layout_passes=False),
    )
    def k(k1_hbm, k2_hbm, out_hbm, spill_hbm, bm, cbufa, cbufb, b0, b1,
          c1_ref, cu_ref, c2_ref, sem0, sem1):
        wid = lax.axis_index("s") * NC + lax.axis_index("c")
        lo = wid * SPAN
        bufs = (b0, b1)
        sems = (sem0, sem1)
        zero = jnp.zeros((L,), jnp.int32)
        sentv = jnp.full((L,), SENT, jnp.int32)
        c1_ref[...] = zero
        cu_ref[...] = zero
        c2_ref[...] = zero
        spill_row = (spill_hbm, wid * SPILL)
        _zero_bitmap(bm)
        _scan_stream(k1_hbm, bm, c1_ref, bufs, sems, cbufa, cbufb, lo,
                     None, sentv)
        total = _scan_stream(k2_hbm, bm, cu_ref, bufs, sems, cbufa,
                             cbufb, lo, spill_row, sentv)
        _zero_bitmap(bm)
        _replay_spill(spill_row, total, bm, c2_ref, bufs, sems)
        for idx, ref in ((0, c1_ref), (1, cu_ref), (2, c2_ref)):
            pltpu.sync_copy(ref, out_hbm.at[idx, wid])

    return k(keys1, keys2)


def _tc_finish(counts):
    def body(c_ref, o_ref):
        c = c_ref[...].astype(jnp.float32)
        s1 = jnp.sum(c[0])
        su = jnp.sum(c[1])
        s2 = jnp.sum(c[2])
        o_ref[...] = jnp.reshape((s2 - su) / (s1 + su), (1, 1))

    out = pl.pallas_call(
        body,
        out_shape=jax.ShapeDtypeStruct((1, 1), jnp.float32),
    )(counts)
    return out.reshape(())


def kernel(edges1, edges2, num_nodes):
    del num_nodes
    keys1, keys2 = _keys_tc(edges1, edges2)
    counts, _ = _sc_counts(keys1, keys2)
    return _tc_finish(counts)


# speedup vs baseline: 2.3943x; 1.0030x over previous
import functools

import jax
import jax.numpy as jnp
from jax import lax
from jax.experimental import pallas as pl
from jax.experimental.pallas import tpu as pltpu
from jax.experimental.pallas import tpu_sc as plsc

N = 10000
E = 320000
KEYSPACE = N * N
NC = 2
NS = 16
NW = NC * NS
L = 16
SPAN = KEYSPACE // NW
WORDS = ((SPAN + 31) // 32 + L - 1) // L * L
CHUNK = 3200
ROUNDS = E // CHUNK
IN_PAIRS = CHUNK // (2 * L)
CAPH = 6000
CBUF = 7808
SPILL = E + 4 * CBUF
SENT = -1

KB = 64000
KG = E // KB


def _keys_tc(edges1, edges2):
    def body(a_ref, b_ref, k1_ref, k2_ref):
        k1_ref[...] = a_ref[0:1, :] * N + a_ref[1:2, :]
        k2_ref[...] = b_ref[0:1, :] * N + b_ref[1:2, :]

    k1, k2 = pl.pallas_call(
        body,
        grid=(KG,),
        in_specs=[
            pl.BlockSpec((2, KB), lambda i: (0, i)),
            pl.BlockSpec((2, KB), lambda i: (0, i)),
        ],
        out_specs=[
            pl.BlockSpec((1, KB), lambda i: (0, i)),
            pl.BlockSpec((1, KB), lambda i: (0, i)),
        ],
        out_shape=[
            jax.ShapeDtypeStruct((1, E), jnp.int32),
            jax.ShapeDtypeStruct((1, E), jnp.int32),
        ],
    )(edges1, edges2)
    return k1.reshape(E), k2.reshape(E)


def _zero_bitmap(bm):
    zeros = jnp.zeros((L,), jnp.int32)

    def body(i, carry):
        bm[pl.ds(i * L, L)] = zeros
        return carry

    lax.fori_loop(0, WORDS // L, body, 0, unroll=8)


def _drain(cbuf, bm, cnt_ref, fill):
    lanes = lax.broadcasted_iota(jnp.int32, (L,), 0)

    def body(j, carry):
        rs = cbuf[pl.ds(j * L, L)]
        valid = (lanes < (fill - j * L)) & (rs != SENT)
        rs = jnp.where(valid, rs, 0)
        w = lax.shift_right_logical(rs, 5)
        bit = lax.shift_left(jnp.int32(1), rs & 31)
        old = plsc.load_gather(bm, [w], mask=valid)
        elig = valid & ((old & bit) == 0)
        _, sel = plsc.scan_count(w, mask=elig)
        sel = sel & elig
        plsc.addupdate_scatter(bm, [w], bit, mask=sel)
        cnt_ref[...] = cnt_ref[...] + jnp.where(sel, 1, 0)
        left = elig & jnp.logical_not(sel)

        @pl.when(jnp.any(left))
        def _():
            def cond(rem):
                return jnp.any(rem)

            def rmw(rem):
                o = plsc.load_gather(bm, [w], mask=rem)
                er = rem & ((o & bit) == 0)
                _, s = plsc.scan_count(w, mask=er)
                s = s & er
                plsc.addupdate_scatter(bm, [w], bit, mask=s)
                cnt_ref[...] = cnt_ref[...] + jnp.where(s, 1, 0)
                return er & jnp.logical_not(s)

            lax.while_loop(cond, rmw, left)

        return carry

    nit = lax.div(fill + (L - 1), L)
    lax.fori_loop(0, nit, body, 0, unroll=False)


def _scan_stream(keys_hbm, bm, cnt_ref, bufs, sems, cbufa, cbufb, lo,
                 spill_row, sentv, goff):
    def start(g, buf, sem):
        ge = g + goff
        ge = jnp.where(ge >= ROUNDS, ge - ROUNDS, ge)
        off = pl.multiple_of(ge * CHUNK, 128)
        pltpu.async_copy(keys_hbm.at[pl.ds(off, CHUNK)], buf, sem)

    def wait(buf, sem):
        src = keys_hbm.at[pl.ds(0, CHUNK)]
        pltpu.make_async_copy(src, buf, sem).wait()

    start(0, bufs[0], sems[0])

    def spill_and_drain(cbuf, fill, cur):
        if spill_row is None:
            _drain(cbuf, bm, cnt_ref, fill)
            return cur
        true16 = jnp.full((L,), True)
        for t in range(8):
            plsc.store_compressed(cbuf.at[pl.ds(fill + t * L, L)], sentv,
                                  mask=true16)
        dst = pl.multiple_of(spill_row[1] + cur, 128)
        pltpu.sync_copy(cbuf, spill_row[0].at[pl.ds(dst, CBUF)])
        _drain(cbuf, bm, cnt_ref, fill)
        return (cur + fill + 127) & ~127

    def compact_chunk(buf, fills):
        def pair_body(i, fills):
            fa, fb = fills
            ka = buf[pl.ds((2 * i) * L, L)]
            kb = buf[pl.ds((2 * i + 1) * L, L)]
            ra = ka - lo
            rb = kb - lo
            inma = plsc.bitcast(ra, jnp.uint32) < jnp.uint32(SPAN)
            inmb = plsc.bitcast(rb, jnp.uint32) < jnp.uint32(SPAN)
            plsc.store_compressed(cbufa.at[pl.ds(fa, L)], ra, mask=inma)
            plsc.store_compressed(cbufb.at[pl.ds(fb, L)], rb, mask=inmb)
            pca = plsc.all_reduce_population_count(inma)
            pcb = plsc.all_reduce_population_count(inmb)
            return fa + pca[0], fb + pcb[0]

        return lax.fori_loop(0, IN_PAIRS, pair_body, fills, unroll=10)

    def chunk_pair(gg, carry):
        for b in (0, 1):
            g = 2 * gg + b
            wait(bufs[b], sems[b])

            @pl.when(g + 1 < ROUNDS)
            def _():
                start(g + 1, bufs[1 - b], sems[1 - b])

            def no_drain(carry):
                return carry

            def do_drain(carry):
                fa, fb, cur = carry
                cur = spill_and_drain(cbufa, fa, cur)
                cur = spill_and_drain(cbufb, fb, cur)
                return jnp.int32(0), jnp.int32(0), cur

            fa, fb, cur = carry
            pred = jnp.maximum(fa, fb) > CAPH - CHUNK // 2
            carry = lax.cond(pred, do_drain, no_drain, (fa, fb, cur))
            fa, fb = compact_chunk(bufs[b], carry[:2])
            carry = (fa, fb, carry[2])
        return carry

    fa, fb, cur = lax.fori_loop(
        0, ROUNDS // 2, chunk_pair,
        (jnp.int32(0), jnp.int32(0), jnp.int32(0)), unroll=False)
    cur = spill_and_drain(cbufa, fa, cur)
    cur = spill_and_drain(cbufb, fb, cur)
    return cur


def _replay_spill(spill_row, total, bm, cnt_ref, bufs, sems):
    hbm, base = spill_row

    def start(j, buf, sem):
        off = pl.multiple_of(base + j * CHUNK, 128)
        pltpu.async_copy(hbm.at[pl.ds(off, CHUNK)], buf, sem)

    def wait(buf, sem):
        src = hbm.at[pl.ds(0, CHUNK)]
        pltpu.make_async_copy(src, buf, sem).wait()

    nchunks = lax.div(total + (CHUNK - 1), CHUNK)

    @pl.when(nchunks > 0)
    def _():
        start(0, bufs[0], sems[0])

        def chunk_pair(jj, carry):
            for b in (0, 1):
                j = 2 * jj + b

                @pl.when(j < nchunks)
                def _():
                    wait(bufs[b], sems[b])

                    @pl.when(j + 1 < nchunks)
                    def _():
                        start(j + 1, bufs[1 - b], sems[1 - b])

                    cnt = jnp.minimum(total - j * CHUNK, CHUNK)
                    _drain(bufs[b], bm, cnt_ref, cnt)

            return carry

        npairs = lax.div(nchunks + 1, 2)
        lax.fori_loop(0, npairs, chunk_pair, 0, unroll=False)


def _sc_counts(keys1, keys2):
    mesh = plsc.VectorSubcoreMesh(
        core_axis_name="c", subcore_axis_name="s", num_cores=NC,
        num_subcores=NS)

    @functools.partial(
        pl.kernel,
        out_type=[
            jax.ShapeDtypeStruct((3, NW, L), jnp.int32),
            jax.ShapeDtypeStruct((NW * SPILL,), jnp.int32),
        ],
        mesh=mesh,
        scratch_types=[
            pltpu.VMEM((WORDS,), jnp.int32),
            pltpu.VMEM((CBUF,), jnp.int32),
            pltpu.VMEM((CBUF,), jnp.int32),
            pltpu.VMEM((CHUNK,), jnp.int32),
            pltpu.VMEM((CHUNK,), jnp.int32),
            pltpu.VMEM((L,), jnp.int32),
            pltpu.VMEM((L,), jnp.int32),
            pltpu.VMEM((L,), jnp.int32),
            pltpu.SemaphoreType.DMA,
            pltpu.SemaphoreType.DMA,
        ],
        compiler_params=pltpu.CompilerParams(needs_layout_passes=False),
    )
    def k(k1_hbm, k2_hbm, out_hbm, spill_hbm, bm, cbufa, cbufb, b0, b1,
          c1_ref, cu_ref, c2_ref, sem0, sem1):
        wid = lax.axis_index("s") * NC + lax.axis_index("c")
        lo = wid * SPAN
        bufs = (b0, b1)
        sems = (sem0, sem1)
        zero = jnp.zeros((L,), jnp.int32)
        sentv = jnp.full((L,), SENT, jnp.int32)
        c1_ref[...] = zero
        cu_ref[...] = zero
        c2_ref[...] = zero
        spill_row = (spill_hbm, wid * SPILL)
        goff = wid * (ROUNDS // NW)
        _zero_bitmap(bm)
        _scan_stream(k1_hbm, bm, c1_ref, bufs, sems, cbufa, cbufb, lo,
                     None, sentv, goff)
        total = _scan_stream(k2_hbm, bm, cu_ref, bufs, sems, cbufa,
                             cbufb, lo, spill_row, sentv, goff)
        _zero_bitmap(bm)
        _replay_spill(spill_row, total, bm, c2_ref, bufs, sems)
        for idx, ref in ((0, c1_ref), (1, cu_ref), (2, c2_ref)):
            pltpu.sync_copy(ref, out_hbm.at[idx, wid])

    return k(keys1, keys2)


def _tc_finish(counts):
    def body(c_ref, o_ref):
        c = c_ref[...].astype(jnp.float32)
        s1 = jnp.sum(c[0])
        su = jnp.sum(c[1])
        s2 = jnp.sum(c[2])
        o_ref[...] = jnp.reshape((s2 - su) / (s1 + su), (1, 1))

    out = pl.pallas_call(
        body,
        out_shape=jax.ShapeDtypeStruct((1, 1), jnp.float32),
    )(counts)
    return out.reshape(())


def kernel(edges1, edges2, num_nodes):
    del num_nodes
    keys1, keys2 = _keys_tc(edges1, edges2)
    counts, _ = _sc_counts(keys1, keys2)
    return _tc_finish(counts)
